# baseline (speedup 1.0000x reference)
import functools

import jax
import jax.numpy as jnp
from jax import lax
from jax.experimental import pallas as pl
from jax.experimental.pallas import tpu as pltpu
from jax.experimental.pallas import tpu_sc as plsc

NC = 2
NS = 16
L = 16
NW = NC * NS

_HALF_PI = 1.5707963267948966
_ATAN_C = (
    0.9999999864226029, -0.33333094234501404, 0.19993058078345885,
    -0.1420715904776521, 0.10654763167310129, -0.07533839991295242,
    0.04304114884132196, -0.016284033210904422, 0.002903795260599931,
)


def _f32(x):
    return jnp.float32(x)


def _rsqrt(x):
    i = lax.bitcast_convert_type(x, jnp.int32)
    i = 0x5F3759DF - lax.shift_right_arithmetic(i, 1)
    y = lax.bitcast_convert_type(i, jnp.float32)
    for _ in range(3):
        y = y * (_f32(1.5) - _f32(0.5) * x * y * y)
    return y


def _atan01(r):
    r2 = r * r
    p = _f32(_ATAN_C[-1])
    for c in _ATAN_C[-2::-1]:
        p = p * r2 + _f32(c)
    return p * r


def _cross(a, b):
    ax, ay, az = a
    bx, by, bz = b
    return (ay * bz - az * by, az * bx - ax * bz, ax * by - ay * bx)


def _qmul(a, b):
    ax, ay, az, aw = a
    bx, by, bz, bw = b
    return (
        aw * bx + bw * ax + (ay * bz - az * by),
        aw * by + bw * ay + (az * bx - ax * bz),
        aw * bz + bw * az + (ax * by - ay * bx),
        aw * bw - (ax * bx + ay * by + az * bz),
    )


def _qrot(q, v):
    qx, qy, qz, qw = q
    ux, uy, uz = _cross((qx, qy, qz), v)
    cx, cy, cz = _cross((qx, qy, qz), (ux, uy, uz))
    two = _f32(2.0)
    return (
        v[0] + two * (qw * ux + cx),
        v[1] + two * (qw * uy + cy),
        v[2] + two * (qw * uz + cz),
    )


def _edge_error(tp, qp, t1, q1, t2, q2):
    qcp = (-qp[0], -qp[1], -qp[2], qp[3])
    qc1 = (-q1[0], -q1[1], -q1[2], q1[3])
    qx, qy, qz, qw = _qmul(_qmul(qcp, qc1), q2)
    dt = (t2[0] - t1[0], t2[1] - t1[1], t2[2] - t1[2])
    u = _qrot(qc1, dt)
    u = (u[0] - tp[0], u[1] - tp[1], u[2] - tp[2])
    tx, ty, tz = _qrot(qcp, u)

    nq2 = qx * qx + qy * qy + qz * qz + qw * qw
    inq = _rsqrt(nq2)
    qx, qy, qz, qw = qx * inq, qy * inq, qz * inq, qw * inq
    neg = qw < _f32(0.0)
    qx = jnp.where(neg, -qx, qx)
    qy = jnp.where(neg, -qy, qy)
    qz = jnp.where(neg, -qz, qz)
    qw = jnp.where(neg, -qw, qw)

    n2 = qx * qx + qy * qy + qz * qz
    inv_n = _rsqrt(jnp.maximum(n2, _f32(1e-30)))
    n = n2 * inv_n

    big = n > qw
    mn = jnp.minimum(n, qw)
    mx = jnp.maximum(n, qw)
    a = _atan01(mn / mx)
    theta = _f32(2.0) * jnp.where(big, _f32(_HALF_PI) - a, a)

    n_small = n < _f32(1e-7)
    scale = jnp.where(
        n_small, _f32(2.0) / jnp.maximum(qw, _f32(1e-12)), theta * inv_n)
    px, py, pz = scale * qx, scale * qy, scale * qz

    th_small = theta < _f32(1e-7)
    ith = _f32(1.0) / jnp.where(th_small, _f32(1.0), theta)
    coef = jnp.where(
        th_small, _f32(1.0 / 12.0), ith * ith - _f32(0.5) * qw * ith * inv_n)

    c1 = _cross((px, py, pz), (tx, ty, tz))
    c2 = _cross((px, py, pz), c1)
    half = _f32(0.5)
    taux = tx - half * c1[0] + coef * c2[0]
    tauy = ty - half * c1[1] + coef * c2[1]
    tauz = tz - half * c1[2] + coef * c2[2]
    return (taux, tauy, tauz, px, py, pz)


def _make_sc_kernel(n_edges, chunk):
    assert n_edges % (NW * chunk) == 0 and chunk % L == 0
    epw = n_edges // NW
    nchunks = epw // chunk
    assert nchunks % 2 == 0
    mesh = plsc.VectorSubcoreMesh(core_axis_name="c", subcore_axis_name="s")
    plane = jax.ShapeDtypeStruct((n_edges,), jnp.float32)

    def _buf():
        return (
            [pltpu.VMEM((chunk,), jnp.int32) for _ in range(2)]
            + [pltpu.VMEM((chunk,), jnp.float32) for _ in range(7)]
            + [pltpu.VMEM((chunk, 8), jnp.float32) for _ in range(2)]
            + [pltpu.VMEM((chunk,), jnp.float32) for _ in range(6)]
            + [pltpu.SemaphoreType.DMA for _ in range(2)]
        )

    @functools.partial(
        pl.kernel,
        mesh=mesh,
        compiler_params=pltpu.CompilerParams(
            needs_layout_passes=False, use_tc_tiling_on_sc=False),
        out_type=(plane,) * 6,
        scratch_types=_buf() + _buf(),
    )
    def k(i1_hbm, i2_hbm, p0, p1, p2, p3, p4, p5, p6, nodes_hbm,
          o0, o1, o2, o3, o4, o5, *scratch):
        p_hbm = (p0, p1, p2, p3, p4, p5, p6)
        o_hbm = (o0, o1, o2, o3, o4, o5)
        nbuf = len(scratch) // 2
        bufs = []
        for b in range(2):
            s = scratch[b * nbuf:(b + 1) * nbuf]
            bufs.append(dict(
                idx=s[0:2], p_v=s[2:9], n_v=s[9:11], o_v=s[11:17],
                sem=s[17:19]))
        wid = lax.axis_index("s") * NC + lax.axis_index("c")
        base0 = wid * epw

        def prefetch(B, c):
            sl = pl.ds(base0 + c * chunk, chunk)
            pltpu.sync_copy(i1_hbm.at[sl], B["idx"][0])
            pltpu.sync_copy(i2_hbm.at[sl], B["idx"][1])
            pltpu.async_copy(
                nodes_hbm.at[B["idx"][0]], B["n_v"][0], B["sem"][0])
            pltpu.async_copy(
                nodes_hbm.at[B["idx"][1]], B["n_v"][1], B["sem"][1])
            for cc in range(7):
                pltpu.sync_copy(p_hbm[cc].at[sl], B["p_v"][cc])

        def wait_gathers(B):
            for j in range(2):
                pltpu.make_async_copy(
                    nodes_hbm.at[B["idx"][j]], B["n_v"][j], B["sem"][j]
                ).wait()

        def compute(B, c):
            p_v, o_v = B["p_v"], B["o_v"]
            n1_v, n2_v = B["n_v"]

            @pl.loop(0, chunk // L)
            def _inner(g):
                rid = lax.iota(jnp.int32, L) + g * L
                gsl = pl.ds(g * L, L)

                def ld2(ref, cc):
                    return plsc.load_gather(
                        ref, [rid, jnp.full((L,), cc, jnp.int32)])

                tp = tuple(p_v[cc][gsl] for cc in range(3))
                qp = tuple(p_v[cc][gsl] for cc in range(3, 7))
                t1 = tuple(ld2(n1_v, cc) for cc in range(3))
                q1 = tuple(ld2(n1_v, cc) for cc in range(3, 7))
                t2 = tuple(ld2(n2_v, cc) for cc in range(3))
                q2 = tuple(ld2(n2_v, cc) for cc in range(3, 7))
                res = _edge_error(tp, qp, t1, q1, t2, q2)
                for cc, val in enumerate(res):
                    o_v[cc][gsl] = val

            sl = pl.ds(base0 + c * chunk, chunk)
            for cc in range(6):
                pltpu.sync_copy(o_v[cc], o_hbm[cc].at[sl])

        prefetch(bufs[0], jnp.int32(0))
        prefetch(bufs[1], jnp.int32(1))
        nc32 = jnp.int32(nchunks)

        @pl.loop(0, nchunks // 2)
        def _outer(it):
            c0 = it * 2
            wait_gathers(bufs[0])
            compute(bufs[0], c0)
            prefetch(bufs[0], lax.rem(c0 + 2, nc32))
            wait_gathers(bufs[1])
            compute(bufs[1], c0 + 1)
            prefetch(bufs[1], lax.rem(c0 + 3, nc32))

        wait_gathers(bufs[0])
        wait_gathers(bufs[1])

    return k


def kernel(edges, poses, nodes):
    n_edges = edges.shape[0]
    i1 = edges[:, 0]
    i2 = edges[:, 1]
    planes = tuple(poses[:, c] for c in range(7))
    nodes8 = jnp.concatenate(
        [nodes, jnp.zeros((nodes.shape[0], 1), nodes.dtype)], axis=1)
    outs = _make_sc_kernel(n_edges, 800)(i1, i2, *planes, nodes8)
    return jnp.stack(outs, axis=-1)


# speedup vs baseline: 1.1796x; 1.1796x over previous
import functools

import jax
import jax.numpy as jnp
from jax import lax
from jax.experimental import pallas as pl
from jax.experimental.pallas import tpu as pltpu
from jax.experimental.pallas import tpu_sc as plsc

NC = 2
NS = 16
L = 16
NW = NC * NS

_HALF_PI = 1.5707963267948966
_ATAN_C = (
    0.9999999864226029, -0.33333094234501404, 0.19993058078345885,
    -0.1420715904776521, 0.10654763167310129, -0.07533839991295242,
    0.04304114884132196, -0.016284033210904422, 0.002903795260599931,
)


def _f32(x):
    return jnp.float32(x)


def _rsqrt(x):
    i = lax.bitcast_convert_type(x, jnp.int32)
    i = 0x5F3759DF - lax.shift_right_arithmetic(i, 1)
    y = lax.bitcast_convert_type(i, jnp.float32)
    for _ in range(3):
        y = y * (_f32(1.5) - _f32(0.5) * x * y * y)
    return y


def _atan01(r):
    r2 = r * r
    p = _f32(_ATAN_C[-1])
    for c in _ATAN_C[-2::-1]:
        p = p * r2 + _f32(c)
    return p * r


def _cross(a, b):
    ax, ay, az = a
    bx, by, bz = b
    return (ay * bz - az * by, az * bx - ax * bz, ax * by - ay * bx)


def _qmul(a, b):
    ax, ay, az, aw = a
    bx, by, bz, bw = b
    return (
        aw * bx + bw * ax + (ay * bz - az * by),
        aw * by + bw * ay + (az * bx - ax * bz),
        aw * bz + bw * az + (ax * by - ay * bx),
        aw * bw - (ax * bx + ay * by + az * bz),
    )


def _qrot(q, v):
    qx, qy, qz, qw = q
    ux, uy, uz = _cross((qx, qy, qz), v)
    cx, cy, cz = _cross((qx, qy, qz), (ux, uy, uz))
    two = _f32(2.0)
    return (
        v[0] + two * (qw * ux + cx),
        v[1] + two * (qw * uy + cy),
        v[2] + two * (qw * uz + cz),
    )


def _edge_error(tp, qp, t1, q1, t2, q2):
    qcp = (-qp[0], -qp[1], -qp[2], qp[3])
    qc1 = (-q1[0], -q1[1], -q1[2], q1[3])
    qx, qy, qz, qw = _qmul(_qmul(qcp, qc1), q2)
    dt = (t2[0] - t1[0], t2[1] - t1[1], t2[2] - t1[2])
    u = _qrot(qc1, dt)
    u = (u[0] - tp[0], u[1] - tp[1], u[2] - tp[2])
    tx, ty, tz = _qrot(qcp, u)

    nq2 = qx * qx + qy * qy + qz * qz + qw * qw
    inq = _rsqrt(nq2)
    qx, qy, qz, qw = qx * inq, qy * inq, qz * inq, qw * inq
    neg = qw < _f32(0.0)
    qx = jnp.where(neg, -qx, qx)
    qy = jnp.where(neg, -qy, qy)
    qz = jnp.where(neg, -qz, qz)
    qw = jnp.where(neg, -qw, qw)

    n2 = qx * qx + qy * qy + qz * qz
    inv_n = _rsqrt(jnp.maximum(n2, _f32(1e-30)))
    n = n2 * inv_n

    big = n > qw
    mn = jnp.minimum(n, qw)
    mx = jnp.maximum(n, qw)
    a = _atan01(mn / mx)
    theta = _f32(2.0) * jnp.where(big, _f32(_HALF_PI) - a, a)

    n_small = n < _f32(1e-7)
    scale = jnp.where(
        n_small, _f32(2.0) / jnp.maximum(qw, _f32(1e-12)), theta * inv_n)
    px, py, pz = scale * qx, scale * qy, scale * qz

    th_small = theta < _f32(1e-7)
    ith = _f32(1.0) / jnp.where(th_small, _f32(1.0), theta)
    coef = jnp.where(
        th_small, _f32(1.0 / 12.0), ith * ith - _f32(0.5) * qw * ith * inv_n)

    c1 = _cross((px, py, pz), (tx, ty, tz))
    c2 = _cross((px, py, pz), c1)
    half = _f32(0.5)
    taux = tx - half * c1[0] + coef * c2[0]
    tauy = ty - half * c1[1] + coef * c2[1]
    tauz = tz - half * c1[2] + coef * c2[2]
    return (taux, tauy, tauz, px, py, pz)


def _make_sc_kernel(n_edges, chunk):
    assert n_edges % (NW * chunk) == 0 and chunk % L == 0
    epw = n_edges // NW
    nchunks = epw // chunk
    assert nchunks % 2 == 0
    mesh = plsc.VectorSubcoreMesh(core_axis_name="c", subcore_axis_name="s")
    plane = jax.ShapeDtypeStruct((n_edges,), jnp.float32)

    def _buf():
        return (
            [pltpu.VMEM((chunk,), jnp.int32) for _ in range(2)]
            + [pltpu.VMEM((chunk,), jnp.float32) for _ in range(7)]
            + [pltpu.VMEM((chunk, 8), jnp.float32) for _ in range(2)]
            + [pltpu.VMEM((chunk,), jnp.float32) for _ in range(6)]
            + [pltpu.SemaphoreType.DMA for _ in range(2)]
        )

    @functools.partial(
        pl.kernel,
        mesh=mesh,
        compiler_params=pltpu.CompilerParams(
            needs_layout_passes=False, use_tc_tiling_on_sc=False),
        out_type=(plane,) * 6,
        scratch_types=_buf() + _buf(),
    )
    def k(i1_hbm, i2_hbm, p0, p1, p2, p3, p4, p5, p6, nodes_hbm,
          o0, o1, o2, o3, o4, o5, *scratch):
        p_hbm = (p0, p1, p2, p3, p4, p5, p6)
        o_hbm = (o0, o1, o2, o3, o4, o5)
        nbuf = len(scratch) // 2
        bufs = []
        for b in range(2):
            s = scratch[b * nbuf:(b + 1) * nbuf]
            bufs.append(dict(
                idx=s[0:2], p_v=s[2:9], n_v=s[9:11], o_v=s[11:17],
                sem=s[17:19]))
        wid = lax.axis_index("s") * NC + lax.axis_index("c")
        base0 = wid * epw

        def prefetch(B, c):
            sl = pl.ds(base0 + c * chunk, chunk)
            pltpu.sync_copy(i1_hbm.at[sl], B["idx"][0])
            pltpu.sync_copy(i2_hbm.at[sl], B["idx"][1])
            pltpu.async_copy(
                nodes_hbm.at[B["idx"][0]], B["n_v"][0], B["sem"][0])
            pltpu.async_copy(
                nodes_hbm.at[B["idx"][1]], B["n_v"][1], B["sem"][1])
            for cc in range(7):
                pltpu.sync_copy(p_hbm[cc].at[sl], B["p_v"][cc])

        def wait_gathers(B):
            for j in range(2):
                pltpu.make_async_copy(
                    nodes_hbm.at[B["idx"][j]], B["n_v"][j], B["sem"][j]
                ).wait()

        def compute(B, c):
            p_v, o_v = B["p_v"], B["o_v"]
            n1_v, n2_v = B["n_v"]

            @pl.loop(0, chunk // L)
            def _inner(g):
                rid = lax.iota(jnp.int32, L) + g * L
                gsl = pl.ds(g * L, L)

                def ld2(ref, cc):
                    return plsc.load_gather(
                        ref, [rid, jnp.full((L,), cc, jnp.int32)])

                tp = tuple(p_v[cc][gsl] for cc in range(3))
                qp = tuple(p_v[cc][gsl] for cc in range(3, 7))
                t1 = tuple(ld2(n1_v, cc) for cc in range(3))
                q1 = tuple(ld2(n1_v, cc) for cc in range(3, 7))
                t2 = tuple(ld2(n2_v, cc) for cc in range(3))
                q2 = tuple(ld2(n2_v, cc) for cc in range(3, 7))
                res = _edge_error(tp, qp, t1, q1, t2, q2)
                for cc, val in enumerate(res):
                    o_v[cc][gsl] = val

            sl = pl.ds(base0 + c * chunk, chunk)
            for cc in range(6):
                pltpu.sync_copy(o_v[cc], o_hbm[cc].at[sl])

        prefetch(bufs[0], jnp.int32(0))
        prefetch(bufs[1], jnp.int32(1))
        nc32 = jnp.int32(nchunks)

        @pl.loop(0, nchunks // 2)
        def _outer(it):
            c0 = it * 2
            wait_gathers(bufs[0])
            compute(bufs[0], c0)
            prefetch(bufs[0], lax.rem(c0 + 2, nc32))
            wait_gathers(bufs[1])
            compute(bufs[1], c0 + 1)
            prefetch(bufs[1], lax.rem(c0 + 3, nc32))

        wait_gathers(bufs[0])
        wait_gathers(bufs[1])

    return k


def kernel(edges, poses, nodes):
    n_edges = edges.shape[0]
    i1 = edges[:, 0]
    i2 = edges[:, 1]
    planes = tuple(poses[:, c] for c in range(7))
    nodes8 = jnp.concatenate(
        [nodes, jnp.zeros((nodes.shape[0], 1), nodes.dtype)], axis=1)
    outs = _make_sc_kernel(n_edges, 2000)(i1, i2, *planes, nodes8)
    return jnp.stack(outs, axis=-1)


# speedup vs baseline: 1.2700x; 1.0766x over previous
import functools

import jax
import jax.numpy as jnp
from jax import lax
from jax.experimental import pallas as pl
from jax.experimental.pallas import tpu as pltpu
from jax.experimental.pallas import tpu_sc as plsc

NC = 2
NS = 16
L = 16
NW = NC * NS

_HALF_PI = 1.5707963267948966
_ATAN_C = (
    0.9999999864226029, -0.33333094234501404, 0.19993058078345885,
    -0.1420715904776521, 0.10654763167310129, -0.07533839991295242,
    0.04304114884132196, -0.016284033210904422, 0.002903795260599931,
)


def _f32(x):
    return jnp.float32(x)


def _rsqrt(x, steps=3):
    i = lax.bitcast_convert_type(x, jnp.int32)
    i = 0x5F3759DF - lax.shift_right_arithmetic(i, 1)
    y = lax.bitcast_convert_type(i, jnp.float32)
    for _ in range(steps):
        y = y * (_f32(1.5) - _f32(0.5) * x * y * y)
    return y


def _atan01(r):
    r2 = r * r
    p = _f32(_ATAN_C[-1])
    for c in _ATAN_C[-2::-1]:
        p = p * r2 + _f32(c)
    return p * r


def _cross(a, b):
    ax, ay, az = a
    bx, by, bz = b
    return (ay * bz - az * by, az * bx - ax * bz, ax * by - ay * bx)


def _qmul(a, b):
    ax, ay, az, aw = a
    bx, by, bz, bw = b
    return (
        aw * bx + bw * ax + (ay * bz - az * by),
        aw * by + bw * ay + (az * bx - ax * bz),
        aw * bz + bw * az + (ax * by - ay * bx),
        aw * bw - (ax * bx + ay * by + az * bz),
    )


def _qrot(q, v):
    qx, qy, qz, qw = q
    ux, uy, uz = _cross((qx, qy, qz), v)
    cx, cy, cz = _cross((qx, qy, qz), (ux, uy, uz))
    two = _f32(2.0)
    return (
        v[0] + two * (qw * ux + cx),
        v[1] + two * (qw * uy + cy),
        v[2] + two * (qw * uz + cz),
    )


def _edge_error(tp, qp, t1, q1, t2, q2):
    qcp = (-qp[0], -qp[1], -qp[2], qp[3])
    qc1 = (-q1[0], -q1[1], -q1[2], q1[3])
    qx, qy, qz, qw = _qmul(_qmul(qcp, qc1), q2)
    dt = (t2[0] - t1[0], t2[1] - t1[1], t2[2] - t1[2])
    u = _qrot(qc1, dt)
    u = (u[0] - tp[0], u[1] - tp[1], u[2] - tp[2])
    tx, ty, tz = _qrot(qcp, u)

    neg = qw < _f32(0.0)
    aw = jnp.abs(qw)

    n2 = qx * qx + qy * qy + qz * qz
    inv_n = _rsqrt(jnp.maximum(n2, _f32(1e-30)), steps=2)
    n = n2 * inv_n

    big = n > aw
    mn = jnp.minimum(n, aw)
    mx = jnp.maximum(n, aw)
    a = _atan01(mn / mx)
    theta = _f32(2.0) * jnp.where(big, _f32(_HALF_PI) - a, a)

    n_small = n < _f32(1e-7)
    scale = jnp.where(
        n_small, _f32(2.0) / jnp.maximum(aw, _f32(1e-12)), theta * inv_n)
    scale = jnp.where(neg, -scale, scale)
    px, py, pz = scale * qx, scale * qy, scale * qz

    th_small = theta < _f32(1e-7)
    ith = _f32(1.0) / jnp.where(th_small, _f32(1.0), theta)
    coef = jnp.where(
        th_small, _f32(1.0 / 12.0), ith * ith - _f32(0.5) * aw * ith * inv_n)

    c1 = _cross((px, py, pz), (tx, ty, tz))
    c2 = _cross((px, py, pz), c1)
    half = _f32(0.5)
    taux = tx - half * c1[0] + coef * c2[0]
    tauy = ty - half * c1[1] + coef * c2[1]
    tauz = tz - half * c1[2] + coef * c2[2]
    return (taux, tauy, tauz, px, py, pz)


def _make_sc_kernel(n_edges, chunk):
    assert n_edges % (NW * chunk) == 0 and chunk % L == 0
    epw = n_edges // NW
    nchunks = epw // chunk
    assert nchunks >= 2
    mesh = plsc.VectorSubcoreMesh(core_axis_name="c", subcore_axis_name="s")
    plane = jax.ShapeDtypeStruct((n_edges,), jnp.float32)

    def _buf():
        return (
            [pltpu.VMEM((chunk,), jnp.int32) for _ in range(2)]
            + [pltpu.VMEM((chunk, 8), jnp.float32) for _ in range(2)]
            + [pltpu.SemaphoreType.DMA for _ in range(2)]
        )

    _shared_scratch = [pltpu.VMEM((chunk,), jnp.float32) for _ in range(13)]

    @functools.partial(
        pl.kernel,
        mesh=mesh,
        compiler_params=pltpu.CompilerParams(
            needs_layout_passes=False, use_tc_tiling_on_sc=False),
        out_type=(plane,) * 6,
        scratch_types=_buf() + _buf() + _shared_scratch,
    )
    def k(i1_hbm, i2_hbm, p0, p1, p2, p3, p4, p5, p6, nodes_hbm,
          o0, o1, o2, o3, o4, o5, *scratch):
        p_hbm = (p0, p1, p2, p3, p4, p5, p6)
        o_hbm = (o0, o1, o2, o3, o4, o5)
        nbuf = (len(scratch) - 13) // 2
        shared = scratch[2 * nbuf:]
        p_v_shared, o_v_shared = shared[:7], shared[7:13]
        bufs = []
        for b in range(2):
            s = scratch[b * nbuf:(b + 1) * nbuf]
            bufs.append(dict(
                idx=s[0:2], p_v=p_v_shared, n_v=s[2:4], o_v=o_v_shared,
                sem=s[4:6]))
        wid = lax.axis_index("s") * NC + lax.axis_index("c")
        base0 = wid * epw

        def prefetch(B, c):
            sl = pl.ds(base0 + c * chunk, chunk)
            pltpu.sync_copy(i1_hbm.at[sl], B["idx"][0])
            pltpu.sync_copy(i2_hbm.at[sl], B["idx"][1])
            pltpu.async_copy(
                nodes_hbm.at[B["idx"][0]], B["n_v"][0], B["sem"][0])
            pltpu.async_copy(
                nodes_hbm.at[B["idx"][1]], B["n_v"][1], B["sem"][1])

        def wait_gathers(B):
            for j in range(2):
                pltpu.make_async_copy(
                    nodes_hbm.at[B["idx"][j]], B["n_v"][j], B["sem"][j]
                ).wait()

        def compute(B, c):
            p_v, o_v = B["p_v"], B["o_v"]
            n1_v, n2_v = B["n_v"]
            psl = pl.ds(base0 + c * chunk, chunk)
            for cc in range(7):
                pltpu.sync_copy(p_hbm[cc].at[psl], p_v[cc])

            @pl.loop(0, chunk // L)
            def _inner(g):
                rid = lax.iota(jnp.int32, L) + g * L
                gsl = pl.ds(g * L, L)

                def ld2(ref, cc):
                    return plsc.load_gather(
                        ref, [rid, jnp.full((L,), cc, jnp.int32)])

                tp = tuple(p_v[cc][gsl] for cc in range(3))
                qp = tuple(p_v[cc][gsl] for cc in range(3, 7))
                t1 = tuple(ld2(n1_v, cc) for cc in range(3))
                q1 = tuple(ld2(n1_v, cc) for cc in range(3, 7))
                t2 = tuple(ld2(n2_v, cc) for cc in range(3))
                q2 = tuple(ld2(n2_v, cc) for cc in range(3, 7))
                res = _edge_error(tp, qp, t1, q1, t2, q2)
                for cc, val in enumerate(res):
                    o_v[cc][gsl] = val

            sl = pl.ds(base0 + c * chunk, chunk)
            for cc in range(6):
                pltpu.sync_copy(o_v[cc], o_hbm[cc].at[sl])

        prefetch(bufs[0], jnp.int32(0))
        prefetch(bufs[1], jnp.int32(1))
        nc32 = jnp.int32(nchunks)

        @pl.loop(0, nchunks // 2)
        def _outer(it):
            c0 = it * 2
            wait_gathers(bufs[0])
            compute(bufs[0], c0)
            prefetch(bufs[0], lax.rem(c0 + 2, nc32))
            wait_gathers(bufs[1])
            compute(bufs[1], c0 + 1)
            prefetch(bufs[1], lax.rem(c0 + 3, nc32))

        wait_gathers(bufs[0])
        if nchunks % 2:
            compute(bufs[0], jnp.int32(nchunks - 1))
        wait_gathers(bufs[1])

    return k


def kernel(edges, poses, nodes):
    n_edges = edges.shape[0]
    i1 = edges[:, 0]
    i2 = edges[:, 1]
    planes = tuple(poses[:, c] for c in range(7))
    nodes8 = jnp.concatenate(
        [nodes, jnp.zeros((nodes.shape[0], 1), nodes.dtype)], axis=1)
    outs = _make_sc_kernel(n_edges, 1600)(i1, i2, *planes, nodes8)
    return jnp.stack(outs, axis=-1)
